# baseline (speedup 1.0000x reference)
import numpy as np
import jax
import jax.numpy as jnp
from jax.experimental import pallas as pl
from jax.experimental.pallas import tpu as pltpu

EMBED_DIM = 256
MIN_Q = 500
MAX_Q = 800
ALPHA = 2.0
TEMP = 0.1
NUM_HEADS = 8
HD = EMBED_DIM // NUM_HEADS
GRID_H = 64
GRID_W = 64
NCAT = GRID_H * GRID_W
NQ = MIN_Q

_TINY = np.float32(np.finfo(np.float32).tiny)
_COL_TILE = 2048
_K_TILE = 1024


def _threefry2x32(k0, k1, x0, x1):
    ks2 = k0 ^ k1 ^ np.uint32(0x1BD11BDA)
    ks = (k0, k1, ks2)
    rots = ((13, 15, 26, 6), (17, 29, 16, 24))
    x0 = x0 + k0
    x1 = x1 + k1
    for g in range(5):
        for r in rots[g % 2]:
            x0 = x0 + x1
            x1 = (x1 << np.uint32(r)) | (x1 >> np.uint32(32 - r))
            x1 = x1 ^ x0
        x0 = x0 + ks[(g + 1) % 3]
        x1 = x1 + ks[(g + 2) % 3] + np.uint32(g + 1)
    return x0, x1


def _dot_t(a, b):
    return jax.lax.dot_general(a, b, (((1,), (1,)), ((), ())),
                               preferred_element_type=jnp.float32)


def _sampler_body(dref, eref, afref, w1xref, w1yref, b1ref, w2ref, b2ref,
                  qipwref, qipbref, wqref, bqref, wkref, bkref, wvref, bvref,
                  woref, boref, qoutref, qposref):
    b = pl.program_id(0)

    z11 = jnp.zeros((1, 1), jnp.uint32)
    bb = z11 + b.astype(jnp.uint32)
    k0, k1 = _threefry2x32(z11 + np.uint32(0), z11 + np.uint32(42), z11, bb)

    logits = dref[0] / np.float32(TEMP)

    run_max = jnp.full((NQ, 1), -jnp.inf, jnp.float32)
    run_arg = jnp.zeros((NQ, 1), jnp.int32)
    n_tiles = NCAT // _COL_TILE
    rowi = jax.lax.broadcasted_iota(jnp.uint32, (NQ, _COL_TILE), 0)
    coli = jax.lax.broadcasted_iota(jnp.uint32, (NQ, _COL_TILE), 1)
    ctr = rowi * np.uint32(NCAT) + coli
    for t in range(n_tiles):
        x1 = ctr if t == 0 else ctr + np.uint32(t * _COL_TILE)
        o0, o1 = _threefry2x32(k0, k1, z11, x1)
        bits = o0 ^ o1
        fb = (bits >> np.uint32(9)) | np.uint32(0x3F800000)
        fl = jax.lax.bitcast_convert_type(fb, jnp.float32) - np.float32(1.0)
        u = fl * (np.float32(1.0) - _TINY) + _TINY
        u = jnp.maximum(_TINY, u)
        v = (logits[:, t * _COL_TILE:(t + 1) * _COL_TILE]
             - jnp.log(-jnp.log(u)))
        tmax = jnp.max(v, axis=1, keepdims=True)
        colabs = (jax.lax.broadcasted_iota(jnp.int32, (NQ, _COL_TILE), 1)
                  + t * _COL_TILE)
        cand = jnp.where(v == tmax, colabs, NCAT)
        targ = jnp.min(cand, axis=1, keepdims=True)
        upd = tmax > run_max
        run_arg = jnp.where(upd, targ, run_arg)
        run_max = jnp.where(upd, tmax, run_max)

    idx = run_arg
    mcol = jnp.bitwise_and(idx, GRID_W - 1)
    mrow = jnp.right_shift(idx, 6)
    xc = mcol.astype(jnp.float32) / np.float32(GRID_W)
    yc = mrow.astype(jnp.float32) / np.float32(GRID_H)

    h = xc * w1xref[0] + yc * w1yref[0] + b1ref[0]
    h = np.float32(0.5) * h * (np.float32(1.0)
                               + jax.lax.erf(h * np.float32(1.0 / np.sqrt(2.0))))
    qpos = _dot_t(h, w2ref[...]) + b2ref[0]

    emem = eref[0]
    colmask = (jnp.bitwise_and(
        jax.lax.broadcasted_iota(jnp.int32, (NCAT, 1), 0), GRID_W - 1)
        != 0).astype(jnp.float32)
    zrow1 = jnp.zeros((1, EMBED_DIM), jnp.float32)
    zrow64 = jnp.zeros((GRID_W, EMBED_DIM), jnp.float32)
    zrow65 = jnp.zeros((GRID_W + 1, EMBED_DIM), jnp.float32)
    ex = jnp.concatenate([zrow1, emem[:-1, :]], axis=0) * colmask
    ey = jnp.concatenate([zrow64, emem[:-GRID_W, :]], axis=0)
    exy = jnp.concatenate([zrow65, emem[:-(GRID_W + 1), :]], axis=0) * colmask
    pmap = (emem + ex + ey + exy) * np.float32(0.25)
    sampled = jnp.zeros((NQ, EMBED_DIM), jnp.float32)
    for kt in range(NCAT // _K_TILE):
        jab = (jax.lax.broadcasted_iota(jnp.int32, (NQ, _K_TILE), 1)
               + kt * _K_TILE)
        s = (jab == idx).astype(jnp.float32)
        sampled = sampled + jax.lax.dot_general(
            s, pmap[kt * _K_TILE:(kt + 1) * _K_TILE, :],
            (((1,), (0,)), ((), ())), preferred_element_type=jnp.float32)

    ai = _dot_t(afref[0], qipwref[...]) + qipbref[0]
    kmat = _dot_t(ai, wkref[...]) + bkref[0]
    vmat = _dot_t(ai, wvref[...]) + bvref[0]
    q = _dot_t(sampled, wqref[...]) + bqref[0]

    scale = np.float32(1.0) / np.float32(np.sqrt(HD))
    heads = []
    for hh in range(NUM_HEADS):
        sl = slice(hh * HD, (hh + 1) * HD)
        qh = q[:, sl]
        kh = kmat[:, sl]
        vh = vmat[:, sl]
        sc = _dot_t(qh, kh) * scale
        sc = sc - jnp.max(sc, axis=1, keepdims=True)
        p = jnp.exp(sc)
        p = p / jnp.sum(p, axis=1, keepdims=True)
        heads.append(jax.lax.dot_general(
            p, vh, (((1,), (0,)), ((), ())),
            preferred_element_type=jnp.float32))
    o = jnp.concatenate(heads, axis=1)
    content = _dot_t(o, woref[...]) + boref[0]

    qoutref[0] = content + qpos
    qposref[0] = qpos


def kernel(density_map, predicted_count, encoder_memory, agent_features,
           pos_w1, pos_b1, pos_w2, pos_b2, qip_w, qip_b,
           wq, bq, wk, bk, wv, bv, wo, bo):
    B = density_map.shape[0]
    d3 = density_map.reshape(B, 1, NCAT)
    w1x = pos_w1[:, 0].reshape(1, 128)
    w1y = pos_w1[:, 1].reshape(1, 128)
    b1r = pos_b1.reshape(1, 128)
    b2r = pos_b2.reshape(1, 256)
    qipbr = qip_b.reshape(1, 256)
    bqr = bq.reshape(1, 256)
    bkr = bk.reshape(1, 256)
    bvr = bv.reshape(1, 256)
    bor = bo.reshape(1, 256)

    full = lambda shape: pl.BlockSpec(shape, lambda b: (0,) * len(shape))
    per_b3 = lambda shape: pl.BlockSpec(shape, lambda b: (b, 0, 0))

    queries, query_pos = pl.pallas_call(
        _sampler_body,
        grid=(B,),
        in_specs=[
            per_b3((1, 1, NCAT)),
            per_b3((1, NCAT, EMBED_DIM)),
            per_b3((1, 256, 64)),
            full((1, 128)), full((1, 128)), full((1, 128)),
            full((256, 128)), full((1, 256)),
            full((256, 64)), full((1, 256)),
            full((256, 256)), full((1, 256)),
            full((256, 256)), full((1, 256)),
            full((256, 256)), full((1, 256)),
            full((256, 256)), full((1, 256)),
        ],
        out_specs=[
            per_b3((1, NQ, EMBED_DIM)),
            per_b3((1, NQ, EMBED_DIM)),
        ],
        out_shape=[
            jax.ShapeDtypeStruct((B, NQ, EMBED_DIM), jnp.float32),
            jax.ShapeDtypeStruct((B, NQ, EMBED_DIM), jnp.float32),
        ],
        compiler_params=pltpu.CompilerParams(
            dimension_semantics=("parallel",)),
    )(d3, encoder_memory, agent_features, w1x, w1y, b1r, pos_w2, b2r,
      qip_w, qipbr, wq, bqr, wk, bkr, wv, bvr, wo, bor)

    num_q = jnp.clip((predicted_count[:, 0] * ALPHA).astype(jnp.int32),
                     MIN_Q, MAX_Q)
    pm = jnp.zeros((B, NQ), dtype=jnp.bool_)
    return queries, query_pos, num_q, pm


# speedup vs baseline: 1.0191x; 1.0191x over previous
import numpy as np
import jax
import jax.numpy as jnp
from jax.experimental import pallas as pl
from jax.experimental.pallas import tpu as pltpu

EMBED_DIM = 256
MIN_Q = 500
MAX_Q = 800
ALPHA = 2.0
TEMP = 0.1
NUM_HEADS = 8
HD = EMBED_DIM // NUM_HEADS
GRID_H = 64
GRID_W = 64
NCAT = GRID_H * GRID_W
NQ = MIN_Q

_TINY = np.float32(np.finfo(np.float32).tiny)
_COL_TILE = 1024
_K_TILE = 1024


def _threefry2x32(k0, k1, x0, x1):
    ks2 = k0 ^ k1 ^ np.uint32(0x1BD11BDA)
    ks = (k0, k1, ks2)
    rots = ((13, 15, 26, 6), (17, 29, 16, 24))
    x0 = x0 + k0
    x1 = x1 + k1
    for g in range(5):
        for r in rots[g % 2]:
            x0 = x0 + x1
            x1 = (x1 << np.uint32(r)) | (x1 >> np.uint32(32 - r))
            x1 = x1 ^ x0
        x0 = x0 + ks[(g + 1) % 3]
        x1 = x1 + ks[(g + 2) % 3] + np.uint32(g + 1)
    return x0, x1


def _dot_t(a, b):
    return jax.lax.dot_general(a, b, (((1,), (1,)), ((), ())),
                               preferred_element_type=jnp.float32)


def _sampler_body(dref, eref, afref, w1xref, w1yref, b1ref, w2ref, b2ref,
                  qipwref, qipbref, wqref, bqref, wkref, bkref, wvref, bvref,
                  woref, boref, qoutref, qposref):
    b = pl.program_id(0)

    z11 = jnp.zeros((1, 1), jnp.uint32)
    bb = z11 + b.astype(jnp.uint32)
    k0, k1 = _threefry2x32(z11 + np.uint32(0), z11 + np.uint32(42), z11, bb)

    logits = dref[0] / np.float32(TEMP)

    run_max = jnp.full((NQ, 1), -jnp.inf, jnp.float32)
    run_arg = jnp.zeros((NQ, 1), jnp.int32)
    n_tiles = NCAT // _COL_TILE
    rowi = jax.lax.broadcasted_iota(jnp.uint32, (NQ, _COL_TILE), 0)
    coli = jax.lax.broadcasted_iota(jnp.uint32, (NQ, _COL_TILE), 1)
    ctr = rowi * np.uint32(NCAT) + coli
    for t in range(n_tiles):
        x1 = ctr if t == 0 else ctr + np.uint32(t * _COL_TILE)
        o0, o1 = _threefry2x32(k0, k1, z11, x1)
        bits = o0 ^ o1
        fb = (bits >> np.uint32(9)) | np.uint32(0x3F800000)
        fl = jax.lax.bitcast_convert_type(fb, jnp.float32) - np.float32(1.0)
        u = fl + _TINY
        v = (logits[:, t * _COL_TILE:(t + 1) * _COL_TILE]
             - jnp.log(-jnp.log(u)))
        tmax = jnp.max(v, axis=1, keepdims=True)
        colabs = (jax.lax.broadcasted_iota(jnp.int32, (NQ, _COL_TILE), 1)
                  + t * _COL_TILE)
        cand = jnp.where(v == tmax, colabs, NCAT)
        targ = jnp.min(cand, axis=1, keepdims=True)
        upd = tmax > run_max
        run_arg = jnp.where(upd, targ, run_arg)
        run_max = jnp.where(upd, tmax, run_max)

    idx = run_arg
    mcol = jnp.bitwise_and(idx, GRID_W - 1)
    mrow = jnp.right_shift(idx, 6)
    xc = mcol.astype(jnp.float32) / np.float32(GRID_W)
    yc = mrow.astype(jnp.float32) / np.float32(GRID_H)

    h = xc * w1xref[0] + yc * w1yref[0] + b1ref[0]
    h = np.float32(0.5) * h * (np.float32(1.0)
                               + jax.lax.erf(h * np.float32(1.0 / np.sqrt(2.0))))
    qpos = _dot_t(h, w2ref[...]) + b2ref[0]

    emem = eref[0]
    colmask = (jnp.bitwise_and(
        jax.lax.broadcasted_iota(jnp.int32, (NCAT, 1), 0), GRID_W - 1)
        != 0).astype(jnp.float32)
    zrow1 = jnp.zeros((1, EMBED_DIM), jnp.float32)
    zrow64 = jnp.zeros((GRID_W, EMBED_DIM), jnp.float32)
    zrow65 = jnp.zeros((GRID_W + 1, EMBED_DIM), jnp.float32)
    ex = jnp.concatenate([zrow1, emem[:-1, :]], axis=0) * colmask
    ey = jnp.concatenate([zrow64, emem[:-GRID_W, :]], axis=0)
    exy = jnp.concatenate([zrow65, emem[:-(GRID_W + 1), :]], axis=0) * colmask
    pmap = (emem + ex + ey + exy) * np.float32(0.25)
    sampled = jnp.zeros((NQ, EMBED_DIM), jnp.float32)
    for kt in range(NCAT // _K_TILE):
        jab = (jax.lax.broadcasted_iota(jnp.int32, (NQ, _K_TILE), 1)
               + kt * _K_TILE)
        s = (jab == idx).astype(jnp.float32)
        sampled = sampled + jax.lax.dot_general(
            s, pmap[kt * _K_TILE:(kt + 1) * _K_TILE, :],
            (((1,), (0,)), ((), ())), preferred_element_type=jnp.float32)

    ai = _dot_t(afref[0], qipwref[...]) + qipbref[0]
    kmat = _dot_t(ai, wkref[...]) + bkref[0]
    vmat = _dot_t(ai, wvref[...]) + bvref[0]
    q = _dot_t(sampled, wqref[...]) + bqref[0]

    scale = np.float32(1.0) / np.float32(np.sqrt(HD))
    heads = []
    for hh in range(NUM_HEADS):
        sl = slice(hh * HD, (hh + 1) * HD)
        qh = q[:, sl]
        kh = kmat[:, sl]
        vh = vmat[:, sl]
        sc = _dot_t(qh, kh) * scale
        sc = sc - jnp.max(sc, axis=1, keepdims=True)
        p = jnp.exp(sc)
        p = p / jnp.sum(p, axis=1, keepdims=True)
        heads.append(jax.lax.dot_general(
            p, vh, (((1,), (0,)), ((), ())),
            preferred_element_type=jnp.float32))
    o = jnp.concatenate(heads, axis=1)
    content = _dot_t(o, woref[...]) + boref[0]

    qoutref[0] = content + qpos
    qposref[0] = qpos


def kernel(density_map, predicted_count, encoder_memory, agent_features,
           pos_w1, pos_b1, pos_w2, pos_b2, qip_w, qip_b,
           wq, bq, wk, bk, wv, bv, wo, bo):
    B = density_map.shape[0]
    d3 = density_map.reshape(B, 1, NCAT)
    w1x = pos_w1[:, 0].reshape(1, 128)
    w1y = pos_w1[:, 1].reshape(1, 128)
    b1r = pos_b1.reshape(1, 128)
    b2r = pos_b2.reshape(1, 256)
    qipbr = qip_b.reshape(1, 256)
    bqr = bq.reshape(1, 256)
    bkr = bk.reshape(1, 256)
    bvr = bv.reshape(1, 256)
    bor = bo.reshape(1, 256)

    full = lambda shape: pl.BlockSpec(shape, lambda b: (0,) * len(shape))
    per_b3 = lambda shape: pl.BlockSpec(shape, lambda b: (b, 0, 0))

    queries, query_pos = pl.pallas_call(
        _sampler_body,
        grid=(B,),
        in_specs=[
            per_b3((1, 1, NCAT)),
            per_b3((1, NCAT, EMBED_DIM)),
            per_b3((1, 256, 64)),
            full((1, 128)), full((1, 128)), full((1, 128)),
            full((256, 128)), full((1, 256)),
            full((256, 64)), full((1, 256)),
            full((256, 256)), full((1, 256)),
            full((256, 256)), full((1, 256)),
            full((256, 256)), full((1, 256)),
            full((256, 256)), full((1, 256)),
        ],
        out_specs=[
            per_b3((1, NQ, EMBED_DIM)),
            per_b3((1, NQ, EMBED_DIM)),
        ],
        out_shape=[
            jax.ShapeDtypeStruct((B, NQ, EMBED_DIM), jnp.float32),
            jax.ShapeDtypeStruct((B, NQ, EMBED_DIM), jnp.float32),
        ],
        compiler_params=pltpu.CompilerParams(
            dimension_semantics=("parallel",)),
    )(d3, encoder_memory, agent_features, w1x, w1y, b1r, pos_w2, b2r,
      qip_w, qipbr, wq, bqr, wk, bkr, wv, bvr, wo, bor)

    num_q = jnp.clip((predicted_count[:, 0] * ALPHA).astype(jnp.int32),
                     MIN_Q, MAX_Q)
    pm = jnp.zeros((B, NQ), dtype=jnp.bool_)
    return queries, query_pos, num_q, pm


# speedup vs baseline: 1.0413x; 1.0218x over previous
import numpy as np
import jax
import jax.numpy as jnp
from jax.experimental import pallas as pl
from jax.experimental.pallas import tpu as pltpu

EMBED_DIM = 256
MIN_Q = 500
MAX_Q = 800
ALPHA = 2.0
TEMP = 0.1
NUM_HEADS = 8
HD = EMBED_DIM // NUM_HEADS
GRID_H = 64
GRID_W = 64
NCAT = GRID_H * GRID_W
NQ = MIN_Q

_TINY = np.float32(np.finfo(np.float32).tiny)
_COL_TILE = 1024
_K_TILE = 1024


def _threefry2x32(k0, k1, x0, x1):
    ks2 = k0 ^ k1 ^ np.uint32(0x1BD11BDA)
    ks = (k0, k1, ks2)
    rots = ((13, 15, 26, 6), (17, 29, 16, 24))
    x0 = x0 + k0
    x1 = x1 + k1
    for g in range(5):
        for r in rots[g % 2]:
            x0 = x0 + x1
            x1 = (x1 << np.uint32(r)) | (x1 >> np.uint32(32 - r))
            x1 = x1 ^ x0
        x0 = x0 + ks[(g + 1) % 3]
        x1 = x1 + (ks[(g + 2) % 3] + np.uint32(g + 1))
    return x0, x1


def _dot_t(a, b):
    return jax.lax.dot_general(a, b, (((1,), (1,)), ((), ())),
                               preferred_element_type=jnp.float32)


def _sampler_body(dref, eref, afref, w1xref, w1yref, b1ref, w2ref, b2ref,
                  qipwref, qipbref, wqref, bqref, wkref, bkref, wvref, bvref,
                  woref, boref, qoutref, qposref):
    b = pl.program_id(0)

    z11 = jnp.zeros((1, 1), jnp.uint32)
    bb = z11 + b.astype(jnp.uint32)
    k0, k1 = _threefry2x32(z11 + np.uint32(0), z11 + np.uint32(42), z11, bb)

    logits = dref[0] / np.float32(TEMP)

    run_max = jnp.full((NQ, 1), -jnp.inf, jnp.float32)
    run_arg = jnp.zeros((NQ, 1), jnp.int32)
    n_tiles = NCAT // _COL_TILE
    rowi = jax.lax.broadcasted_iota(jnp.uint32, (NQ, _COL_TILE), 0)
    coli = jax.lax.broadcasted_iota(jnp.uint32, (NQ, _COL_TILE), 1)
    ctr = rowi * np.uint32(NCAT) + coli
    for t in range(n_tiles):
        x1 = ctr if t == 0 else ctr + np.uint32(t * _COL_TILE)
        o0, o1 = _threefry2x32(k0, k1, z11, x1)
        bits = o0 ^ o1
        fb = (bits >> np.uint32(9)) | np.uint32(0x3F800000)
        fl = jax.lax.bitcast_convert_type(fb, jnp.float32) - np.float32(1.0)
        u = fl + _TINY
        v = (logits[:, t * _COL_TILE:(t + 1) * _COL_TILE]
             - jnp.log(-jnp.log(u)))
        tmax = jnp.max(v, axis=1, keepdims=True)
        colabs = (jax.lax.broadcasted_iota(jnp.int32, (NQ, _COL_TILE), 1)
                  + t * _COL_TILE)
        cand = jnp.where(v == tmax, colabs, NCAT)
        targ = jnp.min(cand, axis=1, keepdims=True)
        upd = tmax > run_max
        run_arg = jnp.where(upd, targ, run_arg)
        run_max = jnp.where(upd, tmax, run_max)

    idx = run_arg
    mcol = jnp.bitwise_and(idx, GRID_W - 1)
    mrow = jnp.right_shift(idx, 6)
    xc = mcol.astype(jnp.float32) / np.float32(GRID_W)
    yc = mrow.astype(jnp.float32) / np.float32(GRID_H)

    h = xc * w1xref[0] + yc * w1yref[0] + b1ref[0]
    h = np.float32(0.5) * h * (np.float32(1.0)
                               + jax.lax.erf(h * np.float32(1.0 / np.sqrt(2.0))))
    qpos = _dot_t(h, w2ref[...]) + b2ref[0]

    emem = eref[0]
    colmask = (jnp.bitwise_and(
        jax.lax.broadcasted_iota(jnp.int32, (NCAT, 1), 0), GRID_W - 1)
        != 0).astype(jnp.float32)
    zrow1 = jnp.zeros((1, EMBED_DIM), jnp.float32)
    zrow64 = jnp.zeros((GRID_W, EMBED_DIM), jnp.float32)
    zrow65 = jnp.zeros((GRID_W + 1, EMBED_DIM), jnp.float32)
    ex = jnp.concatenate([zrow1, emem[:-1, :]], axis=0) * colmask
    ey = jnp.concatenate([zrow64, emem[:-GRID_W, :]], axis=0)
    exy = jnp.concatenate([zrow65, emem[:-(GRID_W + 1), :]], axis=0) * colmask
    pmap = (emem + ex + ey + exy) * np.float32(0.25)
    sampled = jnp.zeros((NQ, EMBED_DIM), jnp.float32)
    for kt in range(NCAT // _K_TILE):
        jab = (jax.lax.broadcasted_iota(jnp.int32, (NQ, _K_TILE), 1)
               + kt * _K_TILE)
        s = (jab == idx).astype(jnp.float32)
        sampled = sampled + jax.lax.dot_general(
            s, pmap[kt * _K_TILE:(kt + 1) * _K_TILE, :],
            (((1,), (0,)), ((), ())), preferred_element_type=jnp.float32)

    ai = _dot_t(afref[0], qipwref[...]) + qipbref[0]
    kmat = _dot_t(ai, wkref[...]) + bkref[0]
    vmat = _dot_t(ai, wvref[...]) + bvref[0]
    q = _dot_t(sampled, wqref[...]) + bqref[0]

    scale = np.float32(1.0) / np.float32(np.sqrt(HD))
    heads = []
    for hh in range(NUM_HEADS):
        sl = slice(hh * HD, (hh + 1) * HD)
        qh = q[:, sl]
        kh = kmat[:, sl]
        vh = vmat[:, sl]
        sc = _dot_t(qh, kh) * scale
        sc = sc - jnp.max(sc, axis=1, keepdims=True)
        p = jnp.exp(sc)
        p = p / jnp.sum(p, axis=1, keepdims=True)
        heads.append(jax.lax.dot_general(
            p, vh, (((1,), (0,)), ((), ())),
            preferred_element_type=jnp.float32))
    o = jnp.concatenate(heads, axis=1)
    content = _dot_t(o, woref[...]) + boref[0]

    qoutref[0] = content + qpos
    qposref[0] = qpos


def kernel(density_map, predicted_count, encoder_memory, agent_features,
           pos_w1, pos_b1, pos_w2, pos_b2, qip_w, qip_b,
           wq, bq, wk, bk, wv, bv, wo, bo):
    B = density_map.shape[0]
    d3 = density_map.reshape(B, 1, NCAT)
    w1x = pos_w1[:, 0].reshape(1, 128)
    w1y = pos_w1[:, 1].reshape(1, 128)
    b1r = pos_b1.reshape(1, 128)
    b2r = pos_b2.reshape(1, 256)
    qipbr = qip_b.reshape(1, 256)
    bqr = bq.reshape(1, 256)
    bkr = bk.reshape(1, 256)
    bvr = bv.reshape(1, 256)
    bor = bo.reshape(1, 256)

    full = lambda shape: pl.BlockSpec(shape, lambda b: (0,) * len(shape))
    per_b3 = lambda shape: pl.BlockSpec(shape, lambda b: (b, 0, 0))

    queries, query_pos = pl.pallas_call(
        _sampler_body,
        grid=(B,),
        in_specs=[
            per_b3((1, 1, NCAT)),
            per_b3((1, NCAT, EMBED_DIM)),
            per_b3((1, 256, 64)),
            full((1, 128)), full((1, 128)), full((1, 128)),
            full((256, 128)), full((1, 256)),
            full((256, 64)), full((1, 256)),
            full((256, 256)), full((1, 256)),
            full((256, 256)), full((1, 256)),
            full((256, 256)), full((1, 256)),
            full((256, 256)), full((1, 256)),
        ],
        out_specs=[
            per_b3((1, NQ, EMBED_DIM)),
            per_b3((1, NQ, EMBED_DIM)),
        ],
        out_shape=[
            jax.ShapeDtypeStruct((B, NQ, EMBED_DIM), jnp.float32),
            jax.ShapeDtypeStruct((B, NQ, EMBED_DIM), jnp.float32),
        ],
        compiler_params=pltpu.CompilerParams(
            dimension_semantics=("parallel",)),
    )(d3, encoder_memory, agent_features, w1x, w1y, b1r, pos_w2, b2r,
      qip_w, qipbr, wq, bqr, wk, bkr, wv, bvr, wo, bor)

    num_q = jnp.clip((predicted_count[:, 0] * ALPHA).astype(jnp.int32),
                     MIN_Q, MAX_Q)
    pm = jnp.zeros((B, NQ), dtype=jnp.bool_)
    return queries, query_pos, num_q, pm


# speedup vs baseline: 1.0478x; 1.0063x over previous
import numpy as np
import jax
import jax.numpy as jnp
from jax.experimental import pallas as pl
from jax.experimental.pallas import tpu as pltpu

EMBED_DIM = 256
MIN_Q = 500
MAX_Q = 800
ALPHA = 2.0
TEMP = 0.1
NUM_HEADS = 8
HD = EMBED_DIM // NUM_HEADS
GRID_H = 64
GRID_W = 64
NCAT = GRID_H * GRID_W
NQ = MIN_Q

_TINY = np.float32(np.finfo(np.float32).tiny)
_COL_TILE = 1024
_K_TILE = 1024


def _threefry2x32(k0, k1, x0, x1):
    ks2 = k0 ^ k1 ^ np.uint32(0x1BD11BDA)
    ks = (k0, k1, ks2)
    rots = ((13, 15, 26, 6), (17, 29, 16, 24))
    for g in range(5):
        for r in rots[g % 2]:
            x0 = x0 + x1
            x1 = (x1 << np.uint32(r)) | (x1 >> np.uint32(32 - r))
            x1 = x1 ^ x0
        x0 = x0 + ks[(g + 1) % 3]
        x1 = x1 + (ks[(g + 2) % 3] + np.uint32(g + 1))
    return x0, x1


def _dot_t(a, b):
    return jax.lax.dot_general(a, b, (((1,), (1,)), ((), ())),
                               preferred_element_type=jnp.float32)


def _sampler_body(dref, eref, afref, w1xref, w1yref, b1ref, w2ref, b2ref,
                  qipwref, qipbref, wqref, bqref, wkref, bkref, wvref, bvref,
                  woref, boref, qoutref, qposref):
    b = pl.program_id(0)

    z11 = jnp.zeros((1, 1), jnp.uint32)
    bb = z11 + b.astype(jnp.uint32)
    k0, k1 = _threefry2x32(z11 + np.uint32(0), z11 + np.uint32(42),
                           z11, bb + np.uint32(42))

    logits = dref[0] / np.float32(TEMP)

    run_max = jnp.full((NQ, 1), -jnp.inf, jnp.float32)
    run_arg = jnp.zeros((NQ, 1), jnp.int32)
    n_tiles = NCAT // _COL_TILE
    rowi = jax.lax.broadcasted_iota(jnp.uint32, (NQ, _COL_TILE), 0)
    coli = jax.lax.broadcasted_iota(jnp.uint32, (NQ, _COL_TILE), 1)
    ctr = rowi * np.uint32(NCAT) + coli
    for t in range(n_tiles):
        x1 = ctr + (k1 + np.uint32(t * _COL_TILE))
        o0, o1 = _threefry2x32(k0, k1, z11 + k0, x1)
        bits = o0 ^ o1
        fb = (bits >> np.uint32(9)) | np.uint32(0x3F800000)
        fl = jax.lax.bitcast_convert_type(fb, jnp.float32) - np.float32(1.0)
        u = fl + _TINY
        v = (logits[:, t * _COL_TILE:(t + 1) * _COL_TILE]
             - jnp.log(-jnp.log(u)))
        tmax = jnp.max(v, axis=1, keepdims=True)
        colabs = (jax.lax.broadcasted_iota(jnp.int32, (NQ, _COL_TILE), 1)
                  + t * _COL_TILE)
        cand = jnp.where(v == tmax, colabs, NCAT)
        targ = jnp.min(cand, axis=1, keepdims=True)
        upd = tmax > run_max
        run_arg = jnp.where(upd, targ, run_arg)
        run_max = jnp.where(upd, tmax, run_max)

    idx = run_arg
    mcol = jnp.bitwise_and(idx, GRID_W - 1)
    mrow = jnp.right_shift(idx, 6)
    xc = mcol.astype(jnp.float32) / np.float32(GRID_W)
    yc = mrow.astype(jnp.float32) / np.float32(GRID_H)

    h = xc * w1xref[0] + yc * w1yref[0] + b1ref[0]
    h = np.float32(0.5) * h * (np.float32(1.0)
                               + jax.lax.erf(h * np.float32(1.0 / np.sqrt(2.0))))
    qpos = _dot_t(h, w2ref[...]) + b2ref[0]

    emem = eref[0]
    colmask = (jnp.bitwise_and(
        jax.lax.broadcasted_iota(jnp.int32, (NCAT, 1), 0), GRID_W - 1)
        != 0).astype(jnp.float32)
    zrow1 = jnp.zeros((1, EMBED_DIM), jnp.float32)
    zrow64 = jnp.zeros((GRID_W, EMBED_DIM), jnp.float32)
    zrow65 = jnp.zeros((GRID_W + 1, EMBED_DIM), jnp.float32)
    ex = jnp.concatenate([zrow1, emem[:-1, :]], axis=0) * colmask
    ey = jnp.concatenate([zrow64, emem[:-GRID_W, :]], axis=0)
    exy = jnp.concatenate([zrow65, emem[:-(GRID_W + 1), :]], axis=0) * colmask
    pmap = (emem + ex + ey + exy) * np.float32(0.25)
    sampled = jnp.zeros((NQ, EMBED_DIM), jnp.float32)
    for kt in range(NCAT // _K_TILE):
        jab = (jax.lax.broadcasted_iota(jnp.int32, (NQ, _K_TILE), 1)
               + kt * _K_TILE)
        s = (jab == idx).astype(jnp.float32)
        sampled = sampled + jax.lax.dot_general(
            s, pmap[kt * _K_TILE:(kt + 1) * _K_TILE, :],
            (((1,), (0,)), ((), ())), preferred_element_type=jnp.float32)

    ai = _dot_t(afref[0], qipwref[...]) + qipbref[0]
    kmat = _dot_t(ai, wkref[...]) + bkref[0]
    vmat = _dot_t(ai, wvref[...]) + bvref[0]
    q = _dot_t(sampled, wqref[...]) + bqref[0]

    scale = np.float32(1.0) / np.float32(np.sqrt(HD))
    heads = []
    for hh in range(NUM_HEADS):
        sl = slice(hh * HD, (hh + 1) * HD)
        qh = q[:, sl]
        kh = kmat[:, sl]
        vh = vmat[:, sl]
        sc = _dot_t(qh, kh) * scale
        sc = sc - jnp.max(sc, axis=1, keepdims=True)
        p = jnp.exp(sc)
        p = p / jnp.sum(p, axis=1, keepdims=True)
        heads.append(jax.lax.dot_general(
            p, vh, (((1,), (0,)), ((), ())),
            preferred_element_type=jnp.float32))
    o = jnp.concatenate(heads, axis=1)
    content = _dot_t(o, woref[...]) + boref[0]

    qoutref[0] = content + qpos
    qposref[0] = qpos


def kernel(density_map, predicted_count, encoder_memory, agent_features,
           pos_w1, pos_b1, pos_w2, pos_b2, qip_w, qip_b,
           wq, bq, wk, bk, wv, bv, wo, bo):
    B = density_map.shape[0]
    d3 = density_map.reshape(B, 1, NCAT)
    w1x = pos_w1[:, 0].reshape(1, 128)
    w1y = pos_w1[:, 1].reshape(1, 128)
    b1r = pos_b1.reshape(1, 128)
    b2r = pos_b2.reshape(1, 256)
    qipbr = qip_b.reshape(1, 256)
    bqr = bq.reshape(1, 256)
    bkr = bk.reshape(1, 256)
    bvr = bv.reshape(1, 256)
    bor = bo.reshape(1, 256)

    full = lambda shape: pl.BlockSpec(shape, lambda b: (0,) * len(shape))
    per_b3 = lambda shape: pl.BlockSpec(shape, lambda b: (b, 0, 0))

    queries, query_pos = pl.pallas_call(
        _sampler_body,
        grid=(B,),
        in_specs=[
            per_b3((1, 1, NCAT)),
            per_b3((1, NCAT, EMBED_DIM)),
            per_b3((1, 256, 64)),
            full((1, 128)), full((1, 128)), full((1, 128)),
            full((256, 128)), full((1, 256)),
            full((256, 64)), full((1, 256)),
            full((256, 256)), full((1, 256)),
            full((256, 256)), full((1, 256)),
            full((256, 256)), full((1, 256)),
            full((256, 256)), full((1, 256)),
        ],
        out_specs=[
            per_b3((1, NQ, EMBED_DIM)),
            per_b3((1, NQ, EMBED_DIM)),
        ],
        out_shape=[
            jax.ShapeDtypeStruct((B, NQ, EMBED_DIM), jnp.float32),
            jax.ShapeDtypeStruct((B, NQ, EMBED_DIM), jnp.float32),
        ],
        compiler_params=pltpu.CompilerParams(
            dimension_semantics=("parallel",)),
    )(d3, encoder_memory, agent_features, w1x, w1y, b1r, pos_w2, b2r,
      qip_w, qipbr, wq, bqr, wk, bkr, wv, bvr, wo, bor)

    num_q = jnp.clip((predicted_count[:, 0] * ALPHA).astype(jnp.int32),
                     MIN_Q, MAX_Q)
    pm = jnp.zeros((B, NQ), dtype=jnp.bool_)
    return queries, query_pos, num_q, pm


# speedup vs baseline: 1.1581x; 1.1052x over previous
import numpy as np
import jax
import jax.numpy as jnp
from jax.experimental import pallas as pl
from jax.experimental.pallas import tpu as pltpu

EMBED_DIM = 256
MIN_Q = 500
MAX_Q = 800
ALPHA = 2.0
TEMP = 0.1
NUM_HEADS = 8
HD = EMBED_DIM // NUM_HEADS
GRID_H = 64
GRID_W = 64
NCAT = GRID_H * GRID_W
NQ = MIN_Q

_TINY = np.float32(np.finfo(np.float32).tiny)
_COL_TILE = 1024
_K_TILE = 1024


def _threefry2x32(k0, k1, x0, x1):
    ks2 = k0 ^ k1 ^ np.uint32(0x1BD11BDA)
    ks = (k0, k1, ks2)
    rots = ((13, 15, 26, 6), (17, 29, 16, 24))
    for g in range(5):
        for r in rots[g % 2]:
            x0 = x0 + x1
            x1 = (x1 << np.uint32(r)) | (x1 >> np.uint32(32 - r))
            x1 = x1 ^ x0
        x0 = x0 + ks[(g + 1) % 3]
        x1 = x1 + (ks[(g + 2) % 3] + np.uint32(g + 1))
    return x0, x1


def _dot_t(a, b):
    return jax.lax.dot_general(a, b, (((1,), (1,)), ((), ())),
                               preferred_element_type=jnp.float32)


_BATCHES_PER_STEP = 2


def _sampler_body(dref, eref, afref, w1xref, w1yref, b1ref, w2ref, b2ref,
                  qipwref, qipbref, wqref, bqref, wkref, bkref, wvref, bvref,
                  woref, boref, qoutref, qposref):
    for bi in range(_BATCHES_PER_STEP):
        _sample_one(bi, dref, eref, afref, w1xref, w1yref, b1ref, w2ref,
                    b2ref, qipwref, qipbref, wqref, bqref, wkref, bkref,
                    wvref, bvref, woref, boref, qoutref, qposref)


def _sample_one(bi, dref, eref, afref, w1xref, w1yref, b1ref, w2ref, b2ref,
                qipwref, qipbref, wqref, bqref, wkref, bkref, wvref, bvref,
                woref, boref, qoutref, qposref):
    b = pl.program_id(0) * _BATCHES_PER_STEP + bi

    z11 = jnp.zeros((1, 1), jnp.uint32)
    bb = z11 + b.astype(jnp.uint32)
    k0, k1 = _threefry2x32(z11 + np.uint32(0), z11 + np.uint32(42),
                           z11, bb + np.uint32(42))

    logits = dref[bi] / np.float32(TEMP)

    run_max = jnp.full((NQ, 1), -jnp.inf, jnp.float32)
    run_arg = jnp.zeros((NQ, 1), jnp.int32)
    n_tiles = NCAT // _COL_TILE
    rowi = jax.lax.broadcasted_iota(jnp.uint32, (NQ, _COL_TILE), 0)
    coli = jax.lax.broadcasted_iota(jnp.uint32, (NQ, _COL_TILE), 1)
    ctr = rowi * np.uint32(NCAT) + coli
    for t in range(n_tiles):
        x1 = ctr + (k1 + np.uint32(t * _COL_TILE))
        o0, o1 = _threefry2x32(k0, k1, z11 + k0, x1)
        bits = o0 ^ o1
        fb = (bits >> np.uint32(9)) | np.uint32(0x3F800000)
        fl = jax.lax.bitcast_convert_type(fb, jnp.float32) - np.float32(1.0)
        u = fl + _TINY
        v = (logits[:, t * _COL_TILE:(t + 1) * _COL_TILE]
             - jnp.log(-jnp.log(u)))
        tmax = jnp.max(v, axis=1, keepdims=True)
        colabs = (jax.lax.broadcasted_iota(jnp.int32, (NQ, _COL_TILE), 1)
                  + t * _COL_TILE)
        cand = jnp.where(v == tmax, colabs, NCAT)
        targ = jnp.min(cand, axis=1, keepdims=True)
        upd = tmax > run_max
        run_arg = jnp.where(upd, targ, run_arg)
        run_max = jnp.where(upd, tmax, run_max)

    idx = run_arg
    mcol = jnp.bitwise_and(idx, GRID_W - 1)
    mrow = jnp.right_shift(idx, 6)
    xc = mcol.astype(jnp.float32) / np.float32(GRID_W)
    yc = mrow.astype(jnp.float32) / np.float32(GRID_H)

    h = xc * w1xref[0] + yc * w1yref[0] + b1ref[0]
    h = np.float32(0.5) * h * (np.float32(1.0)
                               + jax.lax.erf(h * np.float32(1.0 / np.sqrt(2.0))))
    qpos = _dot_t(h, w2ref[...]) + b2ref[0]

    emem = eref[bi]
    colmask = (jnp.bitwise_and(
        jax.lax.broadcasted_iota(jnp.int32, (NCAT, 1), 0), GRID_W - 1)
        != 0).astype(jnp.float32)
    zrow1 = jnp.zeros((1, EMBED_DIM), jnp.float32)
    zrow64 = jnp.zeros((GRID_W, EMBED_DIM), jnp.float32)
    zrow65 = jnp.zeros((GRID_W + 1, EMBED_DIM), jnp.float32)
    ex = jnp.concatenate([zrow1, emem[:-1, :]], axis=0) * colmask
    ey = jnp.concatenate([zrow64, emem[:-GRID_W, :]], axis=0)
    exy = jnp.concatenate([zrow65, emem[:-(GRID_W + 1), :]], axis=0) * colmask
    pmap = (emem + ex + ey + exy) * np.float32(0.25)
    sampled = jnp.zeros((NQ, EMBED_DIM), jnp.float32)
    for kt in range(NCAT // _K_TILE):
        jab = (jax.lax.broadcasted_iota(jnp.int32, (NQ, _K_TILE), 1)
               + kt * _K_TILE)
        s = (jab == idx).astype(jnp.float32)
        sampled = sampled + jax.lax.dot_general(
            s, pmap[kt * _K_TILE:(kt + 1) * _K_TILE, :],
            (((1,), (0,)), ((), ())), preferred_element_type=jnp.float32)

    ai = _dot_t(afref[bi], qipwref[...]) + qipbref[0]
    kmat = _dot_t(ai, wkref[...]) + bkref[0]
    vmat = _dot_t(ai, wvref[...]) + bvref[0]
    q = _dot_t(sampled, wqref[...]) + bqref[0]

    scale = np.float32(1.0) / np.float32(np.sqrt(HD))
    heads = []
    for hh in range(NUM_HEADS):
        sl = slice(hh * HD, (hh + 1) * HD)
        qh = q[:, sl]
        kh = kmat[:, sl]
        vh = vmat[:, sl]
        sc = _dot_t(qh, kh) * scale
        sc = sc - jnp.max(sc, axis=1, keepdims=True)
        p = jnp.exp(sc)
        p = p / jnp.sum(p, axis=1, keepdims=True)
        heads.append(jax.lax.dot_general(
            p, vh, (((1,), (0,)), ((), ())),
            preferred_element_type=jnp.float32))
    o = jnp.concatenate(heads, axis=1)
    content = _dot_t(o, woref[...]) + boref[0]

    qoutref[bi] = content + qpos
    qposref[bi] = qpos


def kernel(density_map, predicted_count, encoder_memory, agent_features,
           pos_w1, pos_b1, pos_w2, pos_b2, qip_w, qip_b,
           wq, bq, wk, bk, wv, bv, wo, bo):
    B = density_map.shape[0]
    d3 = density_map.reshape(B, 1, NCAT)
    w1x = pos_w1[:, 0].reshape(1, 128)
    w1y = pos_w1[:, 1].reshape(1, 128)
    b1r = pos_b1.reshape(1, 128)
    b2r = pos_b2.reshape(1, 256)
    qipbr = qip_b.reshape(1, 256)
    bqr = bq.reshape(1, 256)
    bkr = bk.reshape(1, 256)
    bvr = bv.reshape(1, 256)
    bor = bo.reshape(1, 256)

    full = lambda shape: pl.BlockSpec(shape, lambda b: (0,) * len(shape))
    per_b3 = lambda shape: pl.BlockSpec(shape, lambda b: (b, 0, 0))

    queries, query_pos = pl.pallas_call(
        _sampler_body,
        grid=(B // _BATCHES_PER_STEP,),
        in_specs=[
            per_b3((_BATCHES_PER_STEP, 1, NCAT)),
            per_b3((_BATCHES_PER_STEP, NCAT, EMBED_DIM)),
            per_b3((_BATCHES_PER_STEP, 256, 64)),
            full((1, 128)), full((1, 128)), full((1, 128)),
            full((256, 128)), full((1, 256)),
            full((256, 64)), full((1, 256)),
            full((256, 256)), full((1, 256)),
            full((256, 256)), full((1, 256)),
            full((256, 256)), full((1, 256)),
            full((256, 256)), full((1, 256)),
        ],
        out_specs=[
            per_b3((_BATCHES_PER_STEP, NQ, EMBED_DIM)),
            per_b3((_BATCHES_PER_STEP, NQ, EMBED_DIM)),
        ],
        out_shape=[
            jax.ShapeDtypeStruct((B, NQ, EMBED_DIM), jnp.float32),
            jax.ShapeDtypeStruct((B, NQ, EMBED_DIM), jnp.float32),
        ],
        compiler_params=pltpu.CompilerParams(
            dimension_semantics=("parallel",)),
    )(d3, encoder_memory, agent_features, w1x, w1y, b1r, pos_w2, b2r,
      qip_w, qipbr, wq, bqr, wk, bkr, wv, bvr, wo, bor)

    num_q = jnp.clip((predicted_count[:, 0] * ALPHA).astype(jnp.int32),
                     MIN_Q, MAX_Q)
    pm = jnp.zeros((B, NQ), dtype=jnp.bool_)
    return queries, query_pos, num_q, pm


# speedup vs baseline: 1.2167x; 1.0506x over previous
import numpy as np
import jax
import jax.numpy as jnp
from jax.experimental import pallas as pl
from jax.experimental.pallas import tpu as pltpu

EMBED_DIM = 256
MIN_Q = 500
MAX_Q = 800
ALPHA = 2.0
TEMP = 0.1
NUM_HEADS = 8
HD = EMBED_DIM // NUM_HEADS
GRID_H = 64
GRID_W = 64
NCAT = GRID_H * GRID_W
NQ = MIN_Q

_TINY = np.float32(np.finfo(np.float32).tiny)
_COL_TILE = 1024
_K_TILE = 1024


def _threefry2x32(k0, k1, x0, x1):
    ks2 = k0 ^ k1 ^ np.uint32(0x1BD11BDA)
    ks = (k0, k1, ks2)
    rots = ((13, 15, 26, 6), (17, 29, 16, 24))
    for g in range(5):
        for r in rots[g % 2]:
            x0 = x0 + x1
            x1 = (x1 << np.uint32(r)) | (x1 >> np.uint32(32 - r))
            x1 = x1 ^ x0
        x0 = x0 + ks[(g + 1) % 3]
        x1 = x1 + (ks[(g + 2) % 3] + np.uint32(g + 1))
    return x0, x1


def _dot_t(a, b):
    return jax.lax.dot_general(a, b, (((1,), (1,)), ((), ())),
                               preferred_element_type=jnp.float32)


_BATCHES_PER_STEP = 4


def _sampler_body(dref, eref, afref, w1xref, w1yref, b1ref, w2ref, b2ref,
                  qipwref, qipbref, wqref, bqref, wkref, bkref, wvref, bvref,
                  woref, boref, qoutref, qposref):
    for bi in range(_BATCHES_PER_STEP):
        _sample_one(bi, dref, eref, afref, w1xref, w1yref, b1ref, w2ref,
                    b2ref, qipwref, qipbref, wqref, bqref, wkref, bkref,
                    wvref, bvref, woref, boref, qoutref, qposref)


def _sample_one(bi, dref, eref, afref, w1xref, w1yref, b1ref, w2ref, b2ref,
                qipwref, qipbref, wqref, bqref, wkref, bkref, wvref, bvref,
                woref, boref, qoutref, qposref):
    b = pl.program_id(0) * _BATCHES_PER_STEP + bi

    z11 = jnp.zeros((1, 1), jnp.uint32)
    bb = z11 + b.astype(jnp.uint32)
    k0, k1 = _threefry2x32(z11 + np.uint32(0), z11 + np.uint32(42),
                           z11, bb + np.uint32(42))

    logits = dref[bi] / np.float32(TEMP)

    run_max = jnp.full((NQ, 1), -jnp.inf, jnp.float32)
    run_arg = jnp.zeros((NQ, 1), jnp.int32)
    n_tiles = NCAT // _COL_TILE
    rowi = jax.lax.broadcasted_iota(jnp.uint32, (NQ, _COL_TILE), 0)
    coli = jax.lax.broadcasted_iota(jnp.uint32, (NQ, _COL_TILE), 1)
    ctr = rowi * np.uint32(NCAT) + coli
    for t in range(n_tiles):
        x1 = ctr + (k1 + np.uint32(t * _COL_TILE))
        o0, o1 = _threefry2x32(k0, k1, z11 + k0, x1)
        bits = o0 ^ o1
        fb = (bits >> np.uint32(9)) | np.uint32(0x3F800000)
        fl = jax.lax.bitcast_convert_type(fb, jnp.float32) - np.float32(1.0)
        u = fl + _TINY
        v = (logits[:, t * _COL_TILE:(t + 1) * _COL_TILE]
             - jnp.log(-jnp.log(u)))
        tmax = jnp.max(v, axis=1, keepdims=True)
        colabs = (jax.lax.broadcasted_iota(jnp.int32, (NQ, _COL_TILE), 1)
                  + t * _COL_TILE)
        cand = jnp.where(v == tmax, colabs, NCAT)
        targ = jnp.min(cand, axis=1, keepdims=True)
        upd = tmax > run_max
        run_arg = jnp.where(upd, targ, run_arg)
        run_max = jnp.where(upd, tmax, run_max)

    idx = run_arg
    mcol = jnp.bitwise_and(idx, GRID_W - 1)
    mrow = jnp.right_shift(idx, 6)
    xc = mcol.astype(jnp.float32) / np.float32(GRID_W)
    yc = mrow.astype(jnp.float32) / np.float32(GRID_H)

    h = xc * w1xref[0] + yc * w1yref[0] + b1ref[0]
    h = np.float32(0.5) * h * (np.float32(1.0)
                               + jax.lax.erf(h * np.float32(1.0 / np.sqrt(2.0))))
    qpos = _dot_t(h, w2ref[...]) + b2ref[0]

    emem = eref[bi]
    colmask = (jnp.bitwise_and(
        jax.lax.broadcasted_iota(jnp.int32, (NCAT, 1), 0), GRID_W - 1)
        != 0).astype(jnp.float32)
    zrow1 = jnp.zeros((1, EMBED_DIM), jnp.float32)
    zrow64 = jnp.zeros((GRID_W, EMBED_DIM), jnp.float32)
    zrow65 = jnp.zeros((GRID_W + 1, EMBED_DIM), jnp.float32)
    ex = jnp.concatenate([zrow1, emem[:-1, :]], axis=0) * colmask
    ey = jnp.concatenate([zrow64, emem[:-GRID_W, :]], axis=0)
    exy = jnp.concatenate([zrow65, emem[:-(GRID_W + 1), :]], axis=0) * colmask
    pmap = (emem + ex + ey + exy) * np.float32(0.25)
    sampled = jnp.zeros((NQ, EMBED_DIM), jnp.float32)
    for kt in range(NCAT // _K_TILE):
        jab = (jax.lax.broadcasted_iota(jnp.int32, (NQ, _K_TILE), 1)
               + kt * _K_TILE)
        s = (jab == idx).astype(jnp.float32)
        sampled = sampled + jax.lax.dot_general(
            s, pmap[kt * _K_TILE:(kt + 1) * _K_TILE, :],
            (((1,), (0,)), ((), ())), preferred_element_type=jnp.float32)

    ai = _dot_t(afref[bi], qipwref[...]) + qipbref[0]
    kmat = _dot_t(ai, wkref[...]) + bkref[0]
    vmat = _dot_t(ai, wvref[...]) + bvref[0]
    q = _dot_t(sampled, wqref[...]) + bqref[0]

    scale = np.float32(1.0) / np.float32(np.sqrt(HD))
    heads = []
    for hh in range(NUM_HEADS):
        sl = slice(hh * HD, (hh + 1) * HD)
        qh = q[:, sl]
        kh = kmat[:, sl]
        vh = vmat[:, sl]
        sc = _dot_t(qh, kh) * scale
        sc = sc - jnp.max(sc, axis=1, keepdims=True)
        p = jnp.exp(sc)
        p = p / jnp.sum(p, axis=1, keepdims=True)
        heads.append(jax.lax.dot_general(
            p, vh, (((1,), (0,)), ((), ())),
            preferred_element_type=jnp.float32))
    o = jnp.concatenate(heads, axis=1)
    content = _dot_t(o, woref[...]) + boref[0]

    qoutref[bi] = content + qpos
    qposref[bi] = qpos


def kernel(density_map, predicted_count, encoder_memory, agent_features,
           pos_w1, pos_b1, pos_w2, pos_b2, qip_w, qip_b,
           wq, bq, wk, bk, wv, bv, wo, bo):
    B = density_map.shape[0]
    d3 = density_map.reshape(B, 1, NCAT)
    w1x = pos_w1[:, 0].reshape(1, 128)
    w1y = pos_w1[:, 1].reshape(1, 128)
    b1r = pos_b1.reshape(1, 128)
    b2r = pos_b2.reshape(1, 256)
    qipbr = qip_b.reshape(1, 256)
    bqr = bq.reshape(1, 256)
    bkr = bk.reshape(1, 256)
    bvr = bv.reshape(1, 256)
    bor = bo.reshape(1, 256)

    full = lambda shape: pl.BlockSpec(shape, lambda b: (0,) * len(shape))
    per_b3 = lambda shape: pl.BlockSpec(shape, lambda b: (b, 0, 0))

    queries, query_pos = pl.pallas_call(
        _sampler_body,
        grid=(B // _BATCHES_PER_STEP,),
        in_specs=[
            per_b3((_BATCHES_PER_STEP, 1, NCAT)),
            per_b3((_BATCHES_PER_STEP, NCAT, EMBED_DIM)),
            per_b3((_BATCHES_PER_STEP, 256, 64)),
            full((1, 128)), full((1, 128)), full((1, 128)),
            full((256, 128)), full((1, 256)),
            full((256, 64)), full((1, 256)),
            full((256, 256)), full((1, 256)),
            full((256, 256)), full((1, 256)),
            full((256, 256)), full((1, 256)),
            full((256, 256)), full((1, 256)),
        ],
        out_specs=[
            per_b3((_BATCHES_PER_STEP, NQ, EMBED_DIM)),
            per_b3((_BATCHES_PER_STEP, NQ, EMBED_DIM)),
        ],
        out_shape=[
            jax.ShapeDtypeStruct((B, NQ, EMBED_DIM), jnp.float32),
            jax.ShapeDtypeStruct((B, NQ, EMBED_DIM), jnp.float32),
        ],
        compiler_params=pltpu.CompilerParams(
            dimension_semantics=("parallel",)),
    )(d3, encoder_memory, agent_features, w1x, w1y, b1r, pos_w2, b2r,
      qip_w, qipbr, wq, bqr, wk, bkr, wv, bvr, wo, bor)

    num_q = jnp.clip((predicted_count[:, 0] * ALPHA).astype(jnp.int32),
                     MIN_Q, MAX_Q)
    pm = jnp.zeros((B, NQ), dtype=jnp.bool_)
    return queries, query_pos, num_q, pm


# speedup vs baseline: 1.3005x; 1.0689x over previous
import numpy as np
import jax
import jax.numpy as jnp
from jax.experimental import pallas as pl
from jax.experimental.pallas import tpu as pltpu

EMBED_DIM = 256
MIN_Q = 500
MAX_Q = 800
ALPHA = 2.0
TEMP = 0.1
NUM_HEADS = 8
HD = EMBED_DIM // NUM_HEADS
GRID_H = 64
GRID_W = 64
NCAT = GRID_H * GRID_W
NQ = MIN_Q

_TINY = np.float32(np.finfo(np.float32).tiny)
_COL_TILE = 1024
_K_TILE = 1024


def _threefry2x32(k0, k1, x0, x1):
    ks2 = k0 ^ k1 ^ np.uint32(0x1BD11BDA)
    ks = (k0, k1, ks2)
    rots = ((13, 15, 26, 6), (17, 29, 16, 24))
    for g in range(5):
        for r in rots[g % 2]:
            x0 = x0 + x1
            x1 = (x1 << np.uint32(r)) | (x1 >> np.uint32(32 - r))
            x1 = x1 ^ x0
        x0 = x0 + ks[(g + 1) % 3]
        x1 = x1 + (ks[(g + 2) % 3] + np.uint32(g + 1))
    return x0, x1


def _dot_t(a, b):
    return jax.lax.dot_general(a, b, (((1,), (1,)), ((), ())),
                               preferred_element_type=jnp.float32)


_BATCHES_PER_STEP = 8


def _sampler_body(dref, eref, afref, w1xref, w1yref, b1ref, w2ref, b2ref,
                  qipwref, qipbref, wqref, bqref, wkref, bkref, wvref, bvref,
                  woref, boref, qoutref, qposref):
    for bi in range(_BATCHES_PER_STEP):
        _sample_one(bi, dref, eref, afref, w1xref, w1yref, b1ref, w2ref,
                    b2ref, qipwref, qipbref, wqref, bqref, wkref, bkref,
                    wvref, bvref, woref, boref, qoutref, qposref)


def _sample_one(bi, dref, eref, afref, w1xref, w1yref, b1ref, w2ref, b2ref,
                qipwref, qipbref, wqref, bqref, wkref, bkref, wvref, bvref,
                woref, boref, qoutref, qposref):
    b = pl.program_id(0) * _BATCHES_PER_STEP + bi

    z11 = jnp.zeros((1, 1), jnp.uint32)
    bb = z11 + b.astype(jnp.uint32)
    k0, k1 = _threefry2x32(z11 + np.uint32(0), z11 + np.uint32(42),
                           z11, bb + np.uint32(42))

    logits = dref[bi] / np.float32(TEMP)

    run_max = jnp.full((NQ, 1), -jnp.inf, jnp.float32)
    run_arg = jnp.zeros((NQ, 1), jnp.int32)
    n_tiles = NCAT // _COL_TILE
    rowi = jax.lax.broadcasted_iota(jnp.uint32, (NQ, _COL_TILE), 0)
    coli = jax.lax.broadcasted_iota(jnp.uint32, (NQ, _COL_TILE), 1)
    ctr = rowi * np.uint32(NCAT) + coli
    for t in range(n_tiles):
        x1 = ctr + (k1 + np.uint32(t * _COL_TILE))
        o0, o1 = _threefry2x32(k0, k1, z11 + k0, x1)
        bits = o0 ^ o1
        fb = (bits >> np.uint32(9)) | np.uint32(0x3F800000)
        fl = jax.lax.bitcast_convert_type(fb, jnp.float32) - np.float32(1.0)
        u = fl + _TINY
        v = (logits[:, t * _COL_TILE:(t + 1) * _COL_TILE]
             - jnp.log(-jnp.log(u)))
        tmax = jnp.max(v, axis=1, keepdims=True)
        colabs = (jax.lax.broadcasted_iota(jnp.int32, (NQ, _COL_TILE), 1)
                  + t * _COL_TILE)
        cand = jnp.where(v == tmax, colabs, NCAT)
        targ = jnp.min(cand, axis=1, keepdims=True)
        upd = tmax > run_max
        run_arg = jnp.where(upd, targ, run_arg)
        run_max = jnp.where(upd, tmax, run_max)

    idx = run_arg
    mcol = jnp.bitwise_and(idx, GRID_W - 1)
    mrow = jnp.right_shift(idx, 6)
    xc = mcol.astype(jnp.float32) / np.float32(GRID_W)
    yc = mrow.astype(jnp.float32) / np.float32(GRID_H)

    h = xc * w1xref[0] + yc * w1yref[0] + b1ref[0]
    h = np.float32(0.5) * h * (np.float32(1.0)
                               + jax.lax.erf(h * np.float32(1.0 / np.sqrt(2.0))))
    qpos = _dot_t(h, w2ref[...]) + b2ref[0]

    emem = eref[bi]
    colmask = (jnp.bitwise_and(
        jax.lax.broadcasted_iota(jnp.int32, (NCAT, 1), 0), GRID_W - 1)
        != 0).astype(jnp.float32)
    zrow1 = jnp.zeros((1, EMBED_DIM), jnp.float32)
    zrow64 = jnp.zeros((GRID_W, EMBED_DIM), jnp.float32)
    zrow65 = jnp.zeros((GRID_W + 1, EMBED_DIM), jnp.float32)
    ex = jnp.concatenate([zrow1, emem[:-1, :]], axis=0) * colmask
    ey = jnp.concatenate([zrow64, emem[:-GRID_W, :]], axis=0)
    exy = jnp.concatenate([zrow65, emem[:-(GRID_W + 1), :]], axis=0) * colmask
    pmap = (emem + ex + ey + exy) * np.float32(0.25)
    sampled = jnp.zeros((NQ, EMBED_DIM), jnp.float32)
    for kt in range(NCAT // _K_TILE):
        jab = (jax.lax.broadcasted_iota(jnp.int32, (NQ, _K_TILE), 1)
               + kt * _K_TILE)
        s = (jab == idx).astype(jnp.float32)
        sampled = sampled + jax.lax.dot_general(
            s, pmap[kt * _K_TILE:(kt + 1) * _K_TILE, :],
            (((1,), (0,)), ((), ())), preferred_element_type=jnp.float32)

    ai = _dot_t(afref[bi], qipwref[...]) + qipbref[0]
    kmat = _dot_t(ai, wkref[...]) + bkref[0]
    vmat = _dot_t(ai, wvref[...]) + bvref[0]
    q = _dot_t(sampled, wqref[...]) + bqref[0]

    scale = np.float32(1.0) / np.float32(np.sqrt(HD))
    heads = []
    for hh in range(NUM_HEADS):
        sl = slice(hh * HD, (hh + 1) * HD)
        qh = q[:, sl]
        kh = kmat[:, sl]
        vh = vmat[:, sl]
        sc = _dot_t(qh, kh) * scale
        sc = sc - jnp.max(sc, axis=1, keepdims=True)
        p = jnp.exp(sc)
        p = p / jnp.sum(p, axis=1, keepdims=True)
        heads.append(jax.lax.dot_general(
            p, vh, (((1,), (0,)), ((), ())),
            preferred_element_type=jnp.float32))
    o = jnp.concatenate(heads, axis=1)
    content = _dot_t(o, woref[...]) + boref[0]

    qoutref[bi] = content + qpos
    qposref[bi] = qpos


def kernel(density_map, predicted_count, encoder_memory, agent_features,
           pos_w1, pos_b1, pos_w2, pos_b2, qip_w, qip_b,
           wq, bq, wk, bk, wv, bv, wo, bo):
    B = density_map.shape[0]
    d3 = density_map.reshape(B, 1, NCAT)
    w1x = pos_w1[:, 0].reshape(1, 128)
    w1y = pos_w1[:, 1].reshape(1, 128)
    b1r = pos_b1.reshape(1, 128)
    b2r = pos_b2.reshape(1, 256)
    qipbr = qip_b.reshape(1, 256)
    bqr = bq.reshape(1, 256)
    bkr = bk.reshape(1, 256)
    bvr = bv.reshape(1, 256)
    bor = bo.reshape(1, 256)

    full = lambda shape: pl.BlockSpec(shape, lambda b: (0,) * len(shape))
    per_b3 = lambda shape: pl.BlockSpec(shape, lambda b: (b, 0, 0))

    queries, query_pos = pl.pallas_call(
        _sampler_body,
        grid=(B // _BATCHES_PER_STEP,),
        in_specs=[
            per_b3((_BATCHES_PER_STEP, 1, NCAT)),
            per_b3((_BATCHES_PER_STEP, NCAT, EMBED_DIM)),
            per_b3((_BATCHES_PER_STEP, 256, 64)),
            full((1, 128)), full((1, 128)), full((1, 128)),
            full((256, 128)), full((1, 256)),
            full((256, 64)), full((1, 256)),
            full((256, 256)), full((1, 256)),
            full((256, 256)), full((1, 256)),
            full((256, 256)), full((1, 256)),
            full((256, 256)), full((1, 256)),
        ],
        out_specs=[
            per_b3((_BATCHES_PER_STEP, NQ, EMBED_DIM)),
            per_b3((_BATCHES_PER_STEP, NQ, EMBED_DIM)),
        ],
        out_shape=[
            jax.ShapeDtypeStruct((B, NQ, EMBED_DIM), jnp.float32),
            jax.ShapeDtypeStruct((B, NQ, EMBED_DIM), jnp.float32),
        ],
        compiler_params=pltpu.CompilerParams(
            dimension_semantics=("parallel",)),
    )(d3, encoder_memory, agent_features, w1x, w1y, b1r, pos_w2, b2r,
      qip_w, qipbr, wq, bqr, wk, bkr, wv, bvr, wo, bor)

    num_q = jnp.clip((predicted_count[:, 0] * ALPHA).astype(jnp.int32),
                     MIN_Q, MAX_Q)
    pm = jnp.zeros((B, NQ), dtype=jnp.bool_)
    return queries, query_pos, num_q, pm


# speedup vs baseline: 1.3260x; 1.0196x over previous
import numpy as np
import jax
import jax.numpy as jnp
from jax.experimental import pallas as pl
from jax.experimental.pallas import tpu as pltpu

EMBED_DIM = 256
MIN_Q = 500
MAX_Q = 800
ALPHA = 2.0
TEMP = 0.1
NUM_HEADS = 8
HD = EMBED_DIM // NUM_HEADS
GRID_H = 64
GRID_W = 64
NCAT = GRID_H * GRID_W
NQ = MIN_Q

_TINY = np.float32(np.finfo(np.float32).tiny)
_COL_TILE = 1024
_K_TILE = 1024


def _threefry2x32(k0, k1, x0, x1):
    ks2 = k0 ^ k1 ^ np.uint32(0x1BD11BDA)
    ks = (k0, k1, ks2)
    rots = ((13, 15, 26, 6), (17, 29, 16, 24))
    for g in range(5):
        for r in rots[g % 2]:
            x0 = x0 + x1
            x1 = (x1 << np.uint32(r)) | (x1 >> np.uint32(32 - r))
            x1 = x1 ^ x0
        x0 = x0 + ks[(g + 1) % 3]
        x1 = x1 + (ks[(g + 2) % 3] + np.uint32(g + 1))
    return x0, x1


def _dot_t(a, b):
    return jax.lax.dot_general(a, b, (((1,), (1,)), ((), ())),
                               preferred_element_type=jnp.float32)


_BATCHES_PER_STEP = 8


def _sampler_body(dref, ehbm, afref, w1xref, w1yref, b1ref, w2ref, b2ref,
                  qipwref, qipbref, wqref, bqref, wkref, bkref, wvref, bvref,
                  woref, boref, qoutref, qposref, evref, esem):
    ecopy = pltpu.make_async_copy(
        ehbm.at[:, 0:NCAT, :], evref, esem)
    ecopy.start()
    for bi in range(_BATCHES_PER_STEP):
        _sample_one(bi, dref, evref, afref, w1xref, w1yref, b1ref, w2ref,
                    b2ref, qipwref, qipbref, wqref, bqref, wkref, bkref,
                    wvref, bvref, woref, boref, qoutref, qposref,
                    ecopy if bi == 0 else None)


def _sample_one(bi, dref, eref, afref, w1xref, w1yref, b1ref, w2ref, b2ref,
                qipwref, qipbref, wqref, bqref, wkref, bkref, wvref, bvref,
                woref, boref, qoutref, qposref, ecopy):
    b = pl.program_id(0) * _BATCHES_PER_STEP + bi

    z11 = jnp.zeros((1, 1), jnp.uint32)
    bb = z11 + b.astype(jnp.uint32)
    k0, k1 = _threefry2x32(z11 + np.uint32(0), z11 + np.uint32(42),
                           z11, bb + np.uint32(42))

    logits = dref[bi] / np.float32(TEMP)

    run_max = jnp.full((NQ, 1), -jnp.inf, jnp.float32)
    run_arg = jnp.zeros((NQ, 1), jnp.int32)
    n_tiles = NCAT // _COL_TILE
    rowi = jax.lax.broadcasted_iota(jnp.uint32, (NQ, _COL_TILE), 0)
    coli = jax.lax.broadcasted_iota(jnp.uint32, (NQ, _COL_TILE), 1)
    ctr = rowi * np.uint32(NCAT) + coli
    for t in range(n_tiles):
        x1 = ctr + (k1 + np.uint32(t * _COL_TILE))
        o0, o1 = _threefry2x32(k0, k1, z11 + k0, x1)
        bits = o0 ^ o1
        fb = (bits >> np.uint32(9)) | np.uint32(0x3F800000)
        fl = jax.lax.bitcast_convert_type(fb, jnp.float32) - np.float32(1.0)
        u = fl + _TINY
        v = (logits[:, t * _COL_TILE:(t + 1) * _COL_TILE]
             - jnp.log(-jnp.log(u)))
        tmax = jnp.max(v, axis=1, keepdims=True)
        colabs = (jax.lax.broadcasted_iota(jnp.int32, (NQ, _COL_TILE), 1)
                  + t * _COL_TILE)
        cand = jnp.where(v == tmax, colabs, NCAT)
        targ = jnp.min(cand, axis=1, keepdims=True)
        upd = tmax > run_max
        run_arg = jnp.where(upd, targ, run_arg)
        run_max = jnp.where(upd, tmax, run_max)

    idx = run_arg
    mcol = jnp.bitwise_and(idx, GRID_W - 1)
    mrow = jnp.right_shift(idx, 6)
    xc = mcol.astype(jnp.float32) / np.float32(GRID_W)
    yc = mrow.astype(jnp.float32) / np.float32(GRID_H)

    h = xc * w1xref[0] + yc * w1yref[0] + b1ref[0]
    h = np.float32(0.5) * h * (np.float32(1.0)
                               + jax.lax.erf(h * np.float32(1.0 / np.sqrt(2.0))))
    qpos = _dot_t(h, w2ref[...]) + b2ref[0]

    if ecopy is not None:
        ecopy.wait()
    emem = eref[bi]
    colmask = (jnp.bitwise_and(
        jax.lax.broadcasted_iota(jnp.int32, (NCAT, 1), 0), GRID_W - 1)
        != 0).astype(jnp.float32)
    zrow1 = jnp.zeros((1, EMBED_DIM), jnp.float32)
    zrow64 = jnp.zeros((GRID_W, EMBED_DIM), jnp.float32)
    zrow65 = jnp.zeros((GRID_W + 1, EMBED_DIM), jnp.float32)
    ex = jnp.concatenate([zrow1, emem[:-1, :]], axis=0) * colmask
    ey = jnp.concatenate([zrow64, emem[:-GRID_W, :]], axis=0)
    exy = jnp.concatenate([zrow65, emem[:-(GRID_W + 1), :]], axis=0) * colmask
    pmap = (emem + ex + ey + exy) * np.float32(0.25)
    sampled = jnp.zeros((NQ, EMBED_DIM), jnp.float32)
    for kt in range(NCAT // _K_TILE):
        jab = (jax.lax.broadcasted_iota(jnp.int32, (NQ, _K_TILE), 1)
               + kt * _K_TILE)
        s = (jab == idx).astype(jnp.float32)
        sampled = sampled + jax.lax.dot_general(
            s, pmap[kt * _K_TILE:(kt + 1) * _K_TILE, :],
            (((1,), (0,)), ((), ())), preferred_element_type=jnp.float32)

    ai = _dot_t(afref[bi], qipwref[...]) + qipbref[0]
    kmat = _dot_t(ai, wkref[...]) + bkref[0]
    vmat = _dot_t(ai, wvref[...]) + bvref[0]
    q = _dot_t(sampled, wqref[...]) + bqref[0]

    scale = np.float32(1.0) / np.float32(np.sqrt(HD))
    heads = []
    for hh in range(NUM_HEADS):
        sl = slice(hh * HD, (hh + 1) * HD)
        qh = q[:, sl]
        kh = kmat[:, sl]
        vh = vmat[:, sl]
        sc = _dot_t(qh, kh) * scale
        sc = sc - jnp.max(sc, axis=1, keepdims=True)
        p = jnp.exp(sc)
        p = p / jnp.sum(p, axis=1, keepdims=True)
        heads.append(jax.lax.dot_general(
            p, vh, (((1,), (0,)), ((), ())),
            preferred_element_type=jnp.float32))
    o = jnp.concatenate(heads, axis=1)
    content = _dot_t(o, woref[...]) + boref[0]

    qoutref[bi] = content + qpos
    qposref[bi] = qpos


def kernel(density_map, predicted_count, encoder_memory, agent_features,
           pos_w1, pos_b1, pos_w2, pos_b2, qip_w, qip_b,
           wq, bq, wk, bk, wv, bv, wo, bo):
    B = density_map.shape[0]
    d3 = density_map.reshape(B, 1, NCAT)
    w1x = pos_w1[:, 0].reshape(1, 128)
    w1y = pos_w1[:, 1].reshape(1, 128)
    b1r = pos_b1.reshape(1, 128)
    b2r = pos_b2.reshape(1, 256)
    qipbr = qip_b.reshape(1, 256)
    bqr = bq.reshape(1, 256)
    bkr = bk.reshape(1, 256)
    bvr = bv.reshape(1, 256)
    bor = bo.reshape(1, 256)

    full = lambda shape: pl.BlockSpec(shape, lambda b: (0,) * len(shape))
    per_b3 = lambda shape: pl.BlockSpec(shape, lambda b: (b, 0, 0))

    queries, query_pos = pl.pallas_call(
        _sampler_body,
        grid=(B // _BATCHES_PER_STEP,),
        in_specs=[
            per_b3((_BATCHES_PER_STEP, 1, NCAT)),
            pl.BlockSpec(memory_space=pl.ANY),
            per_b3((_BATCHES_PER_STEP, 256, 64)),
            full((1, 128)), full((1, 128)), full((1, 128)),
            full((256, 128)), full((1, 256)),
            full((256, 64)), full((1, 256)),
            full((256, 256)), full((1, 256)),
            full((256, 256)), full((1, 256)),
            full((256, 256)), full((1, 256)),
            full((256, 256)), full((1, 256)),
        ],
        out_specs=[
            per_b3((_BATCHES_PER_STEP, NQ, EMBED_DIM)),
            per_b3((_BATCHES_PER_STEP, NQ, EMBED_DIM)),
        ],
        out_shape=[
            jax.ShapeDtypeStruct((B, NQ, EMBED_DIM), jnp.float32),
            jax.ShapeDtypeStruct((B, NQ, EMBED_DIM), jnp.float32),
        ],
        scratch_shapes=[
            pltpu.VMEM((B, NCAT, EMBED_DIM), jnp.float32),
            pltpu.SemaphoreType.DMA,
        ],
        compiler_params=pltpu.CompilerParams(
            dimension_semantics=("parallel",)),
    )(d3, encoder_memory, agent_features, w1x, w1y, b1r, pos_w2, b2r,
      qip_w, qipbr, wq, bqr, wk, bkr, wv, bvr, wo, bor)

    num_q = jnp.clip((predicted_count[:, 0] * ALPHA).astype(jnp.int32),
                     MIN_Q, MAX_Q)
    pm = jnp.zeros((B, NQ), dtype=jnp.bool_)
    return queries, query_pos, num_q, pm
